# Initial kernel scaffold; baseline (speedup 1.0000x reference)
import functools

import jax
import jax.numpy as jnp
from jax import lax
from jax.experimental import pallas as pl
from jax.experimental.pallas import tpu as pltpu
from jax.experimental.pallas import tpu_sc as plsc

NU = 50000
D = 64
EDGES = 800000
BATCH = 16384

H = 32
NSUB = 16
NPAD = 51200
EPAD = 819200
ROWS_PT = NPAD // NSUB
EDG_PT = EPAD // NSUB
GCHUNK = 128
NSTREAM = 8
CBLK = GCHUNK * NSTREAM
N_EBLK = EDG_PT // CBLK
EROWS_PT = EDG_PT // GCHUNK
RBLK = 160
N_RBLK = ROWS_PT // RBLK
BPT = BATCH // NSUB

_mesh = plsc.VectorSubcoreMesh(core_axis_name="c", subcore_axis_name="s")

_f32 = jnp.float32
_i32 = jnp.int32


def _degree(deg_rows, deg_vals):
    @functools.partial(
        pl.kernel,
        out_type=jax.ShapeDtypeStruct((2, NPAD), _f32),
        mesh=_mesh,
        scratch_types=[
            pltpu.VMEM_SHARED((NPAD, 16), _f32),
            pltpu.VMEM((NSTREAM, GCHUNK), _i32),
            pltpu.VMEM((NSTREAM, GCHUNK), _f32),
            pltpu.VMEM((GCHUNK, 16), _f32),
            pltpu.VMEM((RBLK, 16), _f32),
            pltpu.VMEM((ROWS_PT,), _f32),
            pltpu.VMEM((RBLK, 16), _f32),
            pltpu.SemaphoreType.DMA,
        ],
    )
    def deg_kernel(rows_hbm, vals_hbm, recip_hbm,
                   acc, rowv, valv, msgv, degv, recv, zbuf, sem):
        cid = lax.axis_index("c")
        sid = lax.axis_index("s")

        @pl.loop(0, RBLK)
        def _(r):
            zbuf[r, :] = jnp.zeros((16,), _f32)

        @pl.loop(0, N_RBLK)
        def _(i):
            pltpu.sync_copy(zbuf, acc.at[pl.ds(sid * ROWS_PT + i * RBLK, RBLK)])

        plsc.subcore_barrier()

        @pl.loop(0, N_EBLK)
        def _(blk):
            b0 = sid * EROWS_PT + blk * NSTREAM
            pltpu.sync_copy(rows_hbm.at[cid].at[pl.ds(b0, NSTREAM)], rowv)
            pltpu.sync_copy(vals_hbm.at[cid].at[pl.ds(b0, NSTREAM)], valv)

            @pl.loop(0, NSTREAM)
            def _(j):
                @pl.loop(0, GCHUNK)
                def _(k):
                    msgv[k, :] = jnp.full((16,), valv[j, k], _f32)

                pltpu.sync_copy(msgv, acc.at[rowv.at[j]], add=True)

        plsc.subcore_barrier()

        iota = lax.iota(_i32, 16)
        zeros16 = jnp.zeros((16,), _i32)

        @pl.loop(0, N_RBLK)
        def _(i):
            r0 = sid * ROWS_PT + i * RBLK
            pltpu.sync_copy(acc.at[pl.ds(r0, RBLK)], degv)

            @pl.loop(0, RBLK // 16)
            def _(g):
                deg16 = plsc.load_gather(degv, [g * 16 + iota, zeros16])
                recv[pl.ds(i * RBLK + g * 16, 16)] = 1.0 / (deg16 + 1e-8)

        pltpu.sync_copy(recv, recip_hbm.at[cid].at[pl.ds(sid * ROWS_PT, ROWS_PT)])

    return deg_kernel(deg_rows, deg_vals)


def _main(sr, sc_, sv, ir, ic, iv, ue_stack, ie_stack, uids2d, iids2d, recip):
    out_type = (
        jax.ShapeDtypeStruct((2, NPAD, H), _f32),
        jax.ShapeDtypeStruct((2, NPAD, H), _f32),
        jax.ShapeDtypeStruct((2, NPAD, H), _f32),
        jax.ShapeDtypeStruct((2, BATCH, 16), _f32),
    )

    @functools.partial(
        pl.kernel,
        out_type=out_type,
        mesh=_mesh,
        scratch_types=[
            pltpu.VMEM_SHARED((NPAD, H), _f32),
            pltpu.VMEM((NSTREAM, GCHUNK), _i32),
            pltpu.VMEM((NSTREAM, GCHUNK), _i32),
            pltpu.VMEM((NSTREAM, GCHUNK), _f32),
            pltpu.VMEM((GCHUNK, H), _f32),
            pltpu.VMEM((GCHUNK, H), _f32),
            pltpu.VMEM((GCHUNK, 16), _f32),
            pltpu.VMEM((RBLK, H), _f32),
            pltpu.VMEM((RBLK, H), _f32),
            pltpu.VMEM((RBLK, H), _f32),
            pltpu.VMEM((RBLK, H), _f32),
            pltpu.VMEM((RBLK,), _f32),
            pltpu.VMEM((RBLK, H), _f32),
            pltpu.SemaphoreType.DMA,
        ],
    )
    def main_kernel(sr_h, sc_h, sv_h, ir_h, ic_h, iv_h,
                    ue_h, ie_h, uid_h, iid_h, rec_h,
                    u_out, h1_out, f_out, part_out,
                    acc, rowv, colv, valv, gathv, iev, prodv,
                    accv, featv, feat2v, outv, recv, zbuf, sem):
        cid = lax.axis_index("c")
        sid = lax.axis_index("s")

        @pl.loop(0, RBLK)
        def _(r):
            zbuf[r, pl.ds(0, 16)] = jnp.zeros((16,), _f32)
            zbuf[r, pl.ds(16, 16)] = jnp.zeros((16,), _f32)

        def zero_acc():
            @pl.loop(0, N_RBLK)
            def _(i):
                pltpu.sync_copy(
                    zbuf, acc.at[pl.ds(sid * ROWS_PT + i * RBLK, RBLK)])

        def spmm(row2d, col2d, val2d, table):
            @pl.loop(0, N_EBLK)
            def _(blk):
                b0 = sid * EROWS_PT + blk * NSTREAM
                pltpu.sync_copy(row2d.at[pl.ds(b0, NSTREAM)], rowv)
                pltpu.sync_copy(col2d.at[pl.ds(b0, NSTREAM)], colv)
                pltpu.sync_copy(val2d.at[pl.ds(b0, NSTREAM)], valv)

                @pl.loop(0, NSTREAM)
                def _(j):
                    pltpu.async_copy(
                        table.at[cid].at[colv.at[j]], gathv, sem).wait()

                    @pl.loop(0, GCHUNK)
                    def _(k):
                        v = valv[j, k]
                        gathv[k, pl.ds(0, 16)] = gathv[k, pl.ds(0, 16)] * v
                        gathv[k, pl.ds(16, 16)] = gathv[k, pl.ds(16, 16)] * v

                    pltpu.sync_copy(gathv, acc.at[rowv.at[j]], add=True)

        def finalize(rec_plane, out_ref, mode, feat_ref=None, feat2_ref=None):
            @pl.loop(0, N_RBLK)
            def _(i):
                r0 = sid * ROWS_PT + i * RBLK
                pltpu.sync_copy(acc.at[pl.ds(r0, RBLK)], accv)
                pltpu.sync_copy(zbuf, acc.at[pl.ds(r0, RBLK)])
                pltpu.sync_copy(rec_h.at[rec_plane].at[pl.ds(r0, RBLK)], recv)
                if feat_ref is not None:
                    pltpu.sync_copy(feat_ref.at[cid].at[pl.ds(r0, RBLK)], featv)
                if feat2_ref is not None:
                    pltpu.sync_copy(feat2_ref.at[cid].at[pl.ds(r0, RBLK)], feat2v)

                @pl.loop(0, RBLK)
                def _(r):
                    s = recv[r]
                    lo = accv[r, pl.ds(0, 16)] * s
                    hi = accv[r, pl.ds(16, 16)] * s
                    if mode == 1:
                        lo = lo + featv[r, pl.ds(0, 16)]
                        hi = hi + featv[r, pl.ds(16, 16)]
                    elif mode == 2:
                        lo = lo + 2.0 * featv[r, pl.ds(0, 16)] + feat2v[r, pl.ds(0, 16)]
                        hi = hi + 2.0 * featv[r, pl.ds(16, 16)] + feat2v[r, pl.ds(16, 16)]
                    outv[r, pl.ds(0, 16)] = lo
                    outv[r, pl.ds(16, 16)] = hi

                pltpu.sync_copy(outv, out_ref.at[cid].at[pl.ds(r0, RBLK)])

        def predict():
            pltpu.sync_copy(uid_h.at[pl.ds(sid * NSTREAM, NSTREAM)], rowv)
            pltpu.sync_copy(iid_h.at[pl.ds(sid * NSTREAM, NSTREAM)], colv)

            @pl.loop(0, NSTREAM)
            def _(j):
                pltpu.async_copy(f_out.at[cid].at[rowv.at[j]], gathv, sem).wait()
                pltpu.async_copy(ie_h.at[cid].at[colv.at[j]], iev, sem).wait()

                @pl.loop(0, GCHUNK)
                def _(k):
                    prodv[k, :] = (
                        gathv[k, pl.ds(0, 16)] * iev[k, pl.ds(0, 16)]
                        + gathv[k, pl.ds(16, 16)] * iev[k, pl.ds(16, 16)])

                pltpu.sync_copy(
                    prodv,
                    part_out.at[cid].at[pl.ds(sid * BPT + j * GCHUNK, GCHUNK)])

        zero_acc()
        plsc.subcore_barrier()
        spmm(ir_h, ic_h, iv_h, ie_h)
        plsc.subcore_barrier()
        finalize(1, u_out, 0)
        plsc.subcore_barrier()
        spmm(sr_h, sc_h, sv_h, ue_h)
        plsc.subcore_barrier()
        finalize(0, h1_out, 1, feat_ref=ue_h)
        plsc.subcore_barrier()
        spmm(sr_h, sc_h, sv_h, h1_out)
        plsc.subcore_barrier()
        finalize(0, f_out, 2, feat_ref=h1_out, feat2_ref=u_out)
        plsc.subcore_barrier()
        predict()

    return main_kernel(sr, sc_, sv, ir, ic, iv, ue_stack, ie_stack,
                       uids2d, iids2d, recip)


def _finish(part):
    def body(p_ref, o_ref):
        s = jnp.sum(p_ref[...], axis=(0, 2))
        o_ref[...] = jax.nn.sigmoid(2.0 * s)[None, :]

    return pl.pallas_call(
        body,
        out_shape=jax.ShapeDtypeStruct((1, BATCH), _f32),
    )(part)


def kernel(user_ids, item_ids, social_row, social_col, social_val,
           info_row, info_col, info_val, user_emb, item_emb):
    def pad_edges(x):
        pad = jnp.zeros((EPAD - EDGES,), x.dtype)
        return jnp.concatenate([x, pad]).reshape(EPAD // GCHUNK, GCHUNK)

    sr = pad_edges(social_row)
    sc_ = pad_edges(social_col)
    sv = pad_edges(social_val)
    ir = pad_edges(info_row)
    ic = pad_edges(info_col)
    iv = pad_edges(info_val)

    ue = jnp.zeros((NPAD, D), _f32).at[:NU].set(user_emb)
    ie = jnp.zeros((NPAD, D), _f32).at[:NU].set(item_emb)
    ue_stack = jnp.stack([ue[:, :H], ue[:, H:]])
    ie_stack = jnp.stack([ie[:, :H], ie[:, H:]])

    uids2d = user_ids.reshape(BATCH // GCHUNK, GCHUNK)
    iids2d = item_ids.reshape(BATCH // GCHUNK, GCHUNK)

    deg_rows = jnp.stack([sr, ir])
    deg_vals = jnp.stack([sv, iv])

    recip = _degree(deg_rows, deg_vals)
    _, _, _, part = _main(sr, sc_, sv, ir, ic, iv,
                          ue_stack, ie_stack, uids2d, iids2d, recip)
    return _finish(part).reshape(BATCH)


# speedup vs baseline: 3.8044x; 3.8044x over previous
import dataclasses
import functools

import jax
import jax.numpy as jnp
from jax import lax
from jax.experimental import pallas as pl
from jax.experimental.pallas import tpu as pltpu
from jax.experimental.pallas import tpu_sc as plsc

NU = 50000
D = 64
EDGES = 800000
BATCH = 16384

H = 32
NSUB = 16
NPAD = 51200
EPAD = 819200
ROWS_PT = NPAD // NSUB
EDG_PT = EPAD // NSUB
GCHUNK = 128
NSTREAM = 8
CBLK = GCHUNK * NSTREAM
N_EBLK = EDG_PT // CBLK
EROWS_PT = EDG_PT // GCHUNK
RBLK = 128
N_RBLK = ROWS_PT // RBLK
NRECROW = NPAD // 128
BPT = BATCH // NSUB

_mesh = plsc.VectorSubcoreMesh(core_axis_name="c", subcore_axis_name="s")

_sc_params = pltpu.CompilerParams(
    needs_layout_passes=False, use_tc_tiling_on_sc=False)

_f32 = jnp.float32
_i32 = jnp.int32


def _degree(deg_rows, deg_vals):
    @functools.partial(
        pl.kernel,
        out_type=jax.ShapeDtypeStruct((2, NRECROW, GCHUNK), _f32),
        mesh=_mesh,
        scratch_types=[
            pltpu.VMEM_SHARED((NPAD, 16), _f32),
            pltpu.VMEM((NSTREAM, GCHUNK), _i32),
            pltpu.VMEM((NSTREAM, GCHUNK), _f32),
            pltpu.VMEM((GCHUNK, 16), _f32),
            pltpu.VMEM((RBLK, 16), _f32),
            pltpu.VMEM((1, GCHUNK), _f32),
            pltpu.VMEM((RBLK, 16), _f32),
            pltpu.SemaphoreType.DMA,
        ],
        compiler_params=_sc_params,
    )
    def deg_kernel(rows_hbm, vals_hbm, recip_hbm,
                   acc, rowv, valv, msgv, degv, recv, zbuf, sem):
        cid = lax.axis_index("c")
        sid = lax.axis_index("s")

        @pl.loop(0, RBLK)
        def _(r):
            zbuf[r, :] = jnp.zeros((16,), _f32)

        @pl.loop(0, N_RBLK)
        def _(i):
            pltpu.sync_copy(zbuf, acc.at[pl.ds(sid * ROWS_PT + i * RBLK, RBLK)])

        plsc.subcore_barrier()

        @pl.loop(0, N_EBLK)
        def _(blk):
            b0 = sid * EROWS_PT + blk * NSTREAM
            pltpu.sync_copy(rows_hbm.at[cid].at[pl.ds(b0, NSTREAM)], rowv)
            pltpu.sync_copy(vals_hbm.at[cid].at[pl.ds(b0, NSTREAM)], valv)

            @pl.loop(0, NSTREAM)
            def _(j):
                @pl.loop(0, GCHUNK // 16)
                def _(g):
                    vvec = valv[j, pl.ds(g * 16, 16)]
                    for i in range(16):
                        msgv[g * 16 + i, :] = jnp.full((16,), vvec[i], _f32)

                pltpu.sync_copy(msgv, acc.at[rowv.at[j]], add=True)

        plsc.subcore_barrier()

        iota = lax.iota(_i32, 16)
        zeros16 = jnp.zeros((16,), _i32)

        @pl.loop(0, N_RBLK)
        def _(i):
            r0 = sid * ROWS_PT + i * RBLK
            pltpu.sync_copy(acc.at[pl.ds(r0, RBLK)], degv)

            @pl.loop(0, RBLK // 16)
            def _(g):
                deg16 = plsc.load_gather(degv, [g * 16 + iota, zeros16])
                recv[0, pl.ds(g * 16, 16)] = 1.0 / (deg16 + 1e-8)

            pltpu.sync_copy(
                recv,
                recip_hbm.at[cid].at[pl.ds(sid * (ROWS_PT // 128) + i, 1)])

    return deg_kernel(deg_rows, deg_vals)


def _main(sr, sc_, sv, ir, ic, iv, ue_stack, ie_stack, uids2d, iids2d, recip):
    out_type = (
        jax.ShapeDtypeStruct((2, NPAD, H), _f32),
        jax.ShapeDtypeStruct((2, NPAD, H), _f32),
        jax.ShapeDtypeStruct((2, NPAD, H), _f32),
        jax.ShapeDtypeStruct((2, BATCH, 16), _f32),
    )

    @functools.partial(
        pl.kernel,
        out_type=out_type,
        mesh=_mesh,
        scratch_types=[
            pltpu.VMEM_SHARED((NPAD, H), _f32),
            pltpu.VMEM((NSTREAM, GCHUNK), _i32),
            pltpu.VMEM((NSTREAM, GCHUNK), _i32),
            pltpu.VMEM((NSTREAM, GCHUNK), _f32),
            pltpu.VMEM((GCHUNK, H), _f32),
            pltpu.VMEM((GCHUNK, 16), _f32),
            pltpu.VMEM((RBLK, H), _f32),
            pltpu.VMEM((RBLK, H), _f32),
            pltpu.VMEM((RBLK, H), _f32),
            pltpu.VMEM((1, GCHUNK), _f32),
            pltpu.VMEM((RBLK, H), _f32),
            pltpu.SemaphoreType.DMA,
        ],
        compiler_params=_sc_params,
    )
    def main_kernel(sr_h, sc_h, sv_h, ir_h, ic_h, iv_h,
                    ue_h, ie_h, uid_h, iid_h, rec_h,
                    u_out, h1_out, f_out, part_out,
                    acc, rowv, colv, valv, gathv, prodv,
                    accv, featv, outv, recv, zbuf, sem):
        iev = accv
        feat2v = gathv
        cid = lax.axis_index("c")
        sid = lax.axis_index("s")

        @pl.loop(0, RBLK)
        def _(r):
            zbuf[r, pl.ds(0, 16)] = jnp.zeros((16,), _f32)
            zbuf[r, pl.ds(16, 16)] = jnp.zeros((16,), _f32)

        def zero_acc():
            @pl.loop(0, N_RBLK)
            def _(i):
                pltpu.sync_copy(
                    zbuf, acc.at[pl.ds(sid * ROWS_PT + i * RBLK, RBLK)])

        def spmm(row2d, col2d, val2d, table):
            @pl.loop(0, N_EBLK)
            def _(blk):
                b0 = sid * EROWS_PT + blk * NSTREAM
                pltpu.sync_copy(row2d.at[pl.ds(b0, NSTREAM)], rowv)
                pltpu.sync_copy(col2d.at[pl.ds(b0, NSTREAM)], colv)
                pltpu.sync_copy(val2d.at[pl.ds(b0, NSTREAM)], valv)

                @pl.loop(0, NSTREAM)
                def _(j):
                    pltpu.async_copy(
                        table.at[cid].at[colv.at[j]], gathv, sem).wait()

                    @pl.loop(0, GCHUNK // 16)
                    def _(g):
                        vvec = valv[j, pl.ds(g * 16, 16)]
                        for i in range(16):
                            k = g * 16 + i
                            v = vvec[i]
                            gathv[k, pl.ds(0, 16)] = gathv[k, pl.ds(0, 16)] * v
                            gathv[k, pl.ds(16, 16)] = gathv[k, pl.ds(16, 16)] * v

                    pltpu.sync_copy(gathv, acc.at[rowv.at[j]], add=True)

        def finalize(rec_plane, out_ref, mode, feat_ref=None, feat2_ref=None):
            @pl.loop(0, N_RBLK)
            def _(i):
                r0 = sid * ROWS_PT + i * RBLK
                pltpu.sync_copy(acc.at[pl.ds(r0, RBLK)], accv)
                pltpu.sync_copy(zbuf, acc.at[pl.ds(r0, RBLK)])
                pltpu.sync_copy(
                    rec_h.at[rec_plane].at[pl.ds(sid * N_RBLK + i, 1)], recv)
                if feat_ref is not None:
                    pltpu.sync_copy(feat_ref.at[cid].at[pl.ds(r0, RBLK)], featv)
                if feat2_ref is not None:
                    pltpu.sync_copy(feat2_ref.at[cid].at[pl.ds(r0, RBLK)], feat2v)

                @pl.loop(0, RBLK // 16)
                def _(g):
                    rvec = recv[0, pl.ds(g * 16, 16)]
                    for i in range(16):
                        r = g * 16 + i
                        s = rvec[i]
                        lo = accv[r, pl.ds(0, 16)] * s
                        hi = accv[r, pl.ds(16, 16)] * s
                        if mode == 1:
                            lo = lo + featv[r, pl.ds(0, 16)]
                            hi = hi + featv[r, pl.ds(16, 16)]
                        elif mode == 2:
                            lo = lo + 2.0 * featv[r, pl.ds(0, 16)] + feat2v[r, pl.ds(0, 16)]
                            hi = hi + 2.0 * featv[r, pl.ds(16, 16)] + feat2v[r, pl.ds(16, 16)]
                        outv[r, pl.ds(0, 16)] = lo
                        outv[r, pl.ds(16, 16)] = hi

                pltpu.sync_copy(outv, out_ref.at[cid].at[pl.ds(r0, RBLK)])

        def predict():
            pltpu.sync_copy(uid_h.at[pl.ds(sid * NSTREAM, NSTREAM)], rowv)
            pltpu.sync_copy(iid_h.at[pl.ds(sid * NSTREAM, NSTREAM)], colv)

            @pl.loop(0, NSTREAM)
            def _(j):
                pltpu.async_copy(f_out.at[cid].at[rowv.at[j]], gathv, sem).wait()
                pltpu.async_copy(ie_h.at[cid].at[colv.at[j]], iev, sem).wait()

                @pl.loop(0, GCHUNK)
                def _(k):
                    prodv[k, :] = (
                        gathv[k, pl.ds(0, 16)] * iev[k, pl.ds(0, 16)]
                        + gathv[k, pl.ds(16, 16)] * iev[k, pl.ds(16, 16)])

                pltpu.sync_copy(
                    prodv,
                    part_out.at[cid].at[pl.ds(sid * BPT + j * GCHUNK, GCHUNK)])

        zero_acc()
        plsc.subcore_barrier()
        spmm(ir_h, ic_h, iv_h, ie_h)
        plsc.subcore_barrier()
        finalize(1, u_out, 0)
        plsc.subcore_barrier()
        spmm(sr_h, sc_h, sv_h, ue_h)
        plsc.subcore_barrier()
        finalize(0, h1_out, 1, feat_ref=ue_h)
        plsc.subcore_barrier()
        spmm(sr_h, sc_h, sv_h, h1_out)
        plsc.subcore_barrier()
        finalize(0, f_out, 2, feat_ref=h1_out, feat2_ref=u_out)
        plsc.subcore_barrier()
        predict()

    return main_kernel(sr, sc_, sv, ir, ic, iv, ue_stack, ie_stack,
                       uids2d, iids2d, recip)


def _finish(part):
    def body(p_ref, o_ref):
        s = jnp.sum(p_ref[...], axis=(0, 2))
        o_ref[...] = jax.nn.sigmoid(2.0 * s)[None, :]

    return pl.pallas_call(
        body,
        out_shape=jax.ShapeDtypeStruct((1, BATCH), _f32),
    )(part)


def kernel(user_ids, item_ids, social_row, social_col, social_val,
           info_row, info_col, info_val, user_emb, item_emb):
    def pad_edges(x):
        pad = jnp.zeros((EPAD - EDGES,), x.dtype)
        return jnp.concatenate([x, pad]).reshape(EPAD // GCHUNK, GCHUNK)

    sr = pad_edges(social_row)
    sc_ = pad_edges(social_col)
    sv = pad_edges(social_val)
    ir = pad_edges(info_row)
    ic = pad_edges(info_col)
    iv = pad_edges(info_val)

    ue = jnp.zeros((NPAD, D), _f32).at[:NU].set(user_emb)
    ie = jnp.zeros((NPAD, D), _f32).at[:NU].set(item_emb)
    ue_stack = jnp.stack([ue[:, :H], ue[:, H:]])
    ie_stack = jnp.stack([ie[:, :H], ie[:, H:]])

    uids2d = user_ids.reshape(BATCH // GCHUNK, GCHUNK)
    iids2d = item_ids.reshape(BATCH // GCHUNK, GCHUNK)

    deg_rows = jnp.stack([sr, ir])
    deg_vals = jnp.stack([sv, iv])

    recip = _degree(deg_rows, deg_vals)
    _, _, _, part = _main(sr, sc_, sv, ir, ic, iv,
                          ue_stack, ie_stack, uids2d, iids2d, recip)
    return _finish(part).reshape(BATCH)
